# baseline (speedup 1.0000x reference)
import jax
import jax.numpy as jnp
from jax.experimental import pallas as pl
from jax.experimental.pallas import tpu as pltpu


def _make_body(C, H, W, n1, n2, n3):
    HW = H * W
    s1a, p1a = 2, 6
    Ho1a = (H + 2 * p1a - 1) // s1a + 1
    Wo1a = (W + 2 * p1a - 1) // s1a + 1
    i_lo = -(-p1a // s1a)
    j_lo = -(-p1a // s1a)
    nr = (H + 1) // 2
    nc = (W + 1) // 2
    nS = nr * nc
    kh1, kw1, ph1, pw1 = 10, 10, 5, 6
    Hp1, Wp1 = Ho1a + 2 * ph1, Wo1a + 2 * pw1
    L1 = Hp1 * Wp1
    Lacc1 = (H - 1) * Wp1 + W
    kh2, kw2, ph2, pw2 = 3, 3, 1, 1
    Hp2, Wp2 = H + 2 * ph2, W + 2 * pw2
    L2 = Hp2 * Wp2
    Lacc2 = (H - 1) * Wp2 + W

    def body(x_ref, xs_ref, w12a_ref, b12a_ref, wp1_ref, b1b_ref,
             w2s_ref, b2b_ref, w3_ref, b3_ref, o_ref,
             z1_ref, z2_ref, a1_ref, a2_ref):
        f32, bf16 = jnp.float32, jnp.bfloat16
        x = x_ref[0]
        xs = xs_ref[0]

        m = jnp.max(x, axis=1, keepdims=True)
        x2 = jnp.concatenate([jnp.broadcast_to(m, (C, HW)), x], axis=0)
        x2s = jnp.concatenate([jnp.broadcast_to(m, (C, nS)), xs], axis=0)
        xin = jnp.concatenate([x2, x2s], axis=1).astype(bf16)
        b12a = b12a_ref[...]
        y12 = jnp.dot(w12a_ref[...], xin, preferred_element_type=f32) + b12a
        y12 = jnp.maximum(y12, 0.0)
        y2a = y12[n1:, :HW]
        y1as = y12[:n1, HW:]

        z1_ref[...] = jnp.zeros_like(z1_ref)
        rb1a = jnp.maximum(b12a[:n1], 0.0).astype(bf16)
        for r in range(Ho1a):
            st = (ph1 + r) * Wp1 + pw1
            z1_ref[:n1, st:st + Wo1a] = jnp.broadcast_to(rb1a, (n1, Wo1a))
        for r in range(nr):
            st = (ph1 + i_lo + r) * Wp1 + (pw1 + j_lo)
            z1_ref[:n1, st:st + nc] = y1as[:, r * nc:(r + 1) * nc].astype(bf16)
        z1_ref[n1:, :L1 - kw1 // 2] = z1_ref[:n1, kw1 // 2:]

        a1_ref[...] = jnp.broadcast_to(b1b_ref[...], (n1, Lacc1))
        for di in range(kh1):
            r0 = max(0, ph1 - di)
            r1 = min(H - 1, ph1 + Ho1a - 1 - di)
            o0 = r0 * Wp1
            nd = (r1 - r0) * Wp1 + W
            base = o0 + di * Wp1
            p = jnp.dot(wp1_ref[di * 5], z1_ref[:, base:base + nd],
                        preferred_element_type=f32)
            for dj in range(1, kw1 // 2):
                p = p + jnp.dot(wp1_ref[di * 5 + dj],
                                z1_ref[:, base + dj:base + dj + nd],
                                preferred_element_type=f32)
            a1_ref[:, o0:o0 + nd] = a1_ref[:, o0:o0 + nd] + p
        h1 = jnp.maximum(a1_ref[...], 0.0)

        z2_ref[...] = jnp.zeros_like(z2_ref)
        for r in range(H):
            st = (ph2 + r) * Wp2 + pw2
            z2_ref[:n2, st:st + W] = y2a[:, r * W:(r + 1) * W].astype(bf16)
        z2_ref[n2:2 * n2, :L2 - 1] = z2_ref[:n2, 1:]
        z2_ref[2 * n2:, :L2 - 2] = z2_ref[:n2, 2:]

        a2_ref[...] = jnp.broadcast_to(b2b_ref[...], (n2, Lacc2))
        for di in range(kh2):
            r0 = max(0, ph2 - di)
            r1 = min(H - 1, ph2 + H - 1 - di)
            o0 = r0 * Wp2
            nd = (r1 - r0) * Wp2 + W
            a2_ref[:, o0:o0 + nd] = a2_ref[:, o0:o0 + nd] + jnp.dot(
                w2s_ref[di], z2_ref[:, o0 + di * Wp2:o0 + di * Wp2 + nd],
                preferred_element_type=f32)
        h2 = jnp.maximum(a2_ref[...], 0.0)

        h1v = jnp.concatenate([h1[:, r * Wp1:r * Wp1 + W] for r in range(H)],
                              axis=1)
        h2v = jnp.concatenate([h2[:, r * Wp2:r * Wp2 + W] for r in range(H)],
                              axis=1)
        cat = jnp.concatenate([h1v, h2v], axis=0).astype(bf16)
        h3 = jnp.dot(w3_ref[...], cat, preferred_element_type=f32) + b3_ref[...]
        o_ref[0, :n3] = jnp.maximum(h3, 0.0)
        o_ref[0, n3:] = x

    geom = dict(HW=HW, nS=nS, L1=L1, L2=L2, Lacc1=Lacc1, Lacc2=Lacc2,
                kh1=kh1, kw1=kw1, kh2=kh2, kw2=kw2)
    return body, geom


def kernel(x, w1a, b1a, w1b, b1b, w2a, b2a, w2b, b2b, w3, b3):
    B, C, H, W = x.shape
    n1, n2, n3 = w1b.shape[0], w2b.shape[0], w3.shape[0]
    body, g = _make_body(C, H, W, n1, n2, n3)
    HW, nS, L1, L2 = g["HW"], g["nS"], g["L1"], g["L2"]
    bf16, f32 = jnp.bfloat16, jnp.float32

    x_cm = x.reshape(B, C, HW)
    xs = x[:, :, ::2, ::2].reshape(B, C, nS)
    w12a = jnp.concatenate([w1a.reshape(n1, 2 * C),
                            w2a.reshape(n2, 2 * C)], axis=0).astype(bf16)
    b12a = jnp.concatenate([b1a, b2a]).reshape(n1 + n2, 1).astype(f32)
    w1t = w1b.transpose(2, 3, 0, 1)
    wp1 = jnp.stack([jnp.concatenate([w1t[di, dj], w1t[di, dj + 5]], axis=1)
                     for di in range(g["kh1"]) for dj in range(g["kw1"] // 2)]
                    ).astype(bf16)
    w2t = w2b.transpose(2, 3, 0, 1)
    w2s = jnp.stack([jnp.concatenate([w2t[di, 0], w2t[di, 1], w2t[di, 2]],
                                     axis=1) for di in range(g["kh2"])]
                    ).astype(bf16)
    w3m = w3.reshape(n3, n1 + n2).astype(bf16)
    b1bm = b1b.reshape(n1, 1).astype(f32)
    b2bm = b2b.reshape(n2, 1).astype(f32)
    b3m = b3.reshape(n3, 1).astype(f32)

    out = pl.pallas_call(
        body,
        out_shape=jax.ShapeDtypeStruct((B, n3 + C, HW), f32),
        grid=(B,),
        in_specs=[
            pl.BlockSpec((1, C, HW), lambda b: (b, 0, 0)),
            pl.BlockSpec((1, C, nS), lambda b: (b, 0, 0)),
            pl.BlockSpec((n1 + n2, 2 * C), lambda b: (0, 0)),
            pl.BlockSpec((n1 + n2, 1), lambda b: (0, 0)),
            pl.BlockSpec((50, n1, 2 * n1), lambda b: (0, 0, 0)),
            pl.BlockSpec((n1, 1), lambda b: (0, 0)),
            pl.BlockSpec((3, n2, 3 * n2), lambda b: (0, 0, 0)),
            pl.BlockSpec((n2, 1), lambda b: (0, 0)),
            pl.BlockSpec((n3, n1 + n2), lambda b: (0, 0)),
            pl.BlockSpec((n3, 1), lambda b: (0, 0)),
        ],
        out_specs=pl.BlockSpec((1, n3 + C, HW), lambda b: (b, 0, 0)),
        scratch_shapes=[pltpu.VMEM((2 * n1, L1), bf16),
                        pltpu.VMEM((3 * n2, L2), bf16),
                        pltpu.VMEM((n1, g["Lacc1"]), f32),
                        pltpu.VMEM((n2, g["Lacc2"]), f32)],
        compiler_params=pltpu.CompilerParams(dimension_semantics=("parallel",)),
    )(x_cm, xs, w12a, b12a, wp1, b1bm, w2s, b2bm, w3m, b3m)
    return out.reshape(B, n3 + C, H, W)


# speedup vs baseline: 1.1443x; 1.1443x over previous
import jax
import jax.numpy as jnp
from jax.experimental import pallas as pl
from jax.experimental.pallas import tpu as pltpu


def _make_body(C, H, W, n1, n2, n3):
    HW = H * W
    s1a, p1a = 2, 6
    Ho1a = (H + 2 * p1a - 1) // s1a + 1
    Wo1a = (W + 2 * p1a - 1) // s1a + 1
    i_lo = -(-p1a // s1a)
    j_lo = -(-p1a // s1a)
    nr = (H + 1) // 2
    nc = (W + 1) // 2
    nS = nr * nc
    kh1, kw1, ph1, pw1 = 10, 10, 5, 6
    Hp1, Wp1 = Ho1a + 2 * ph1, Wo1a + 2 * pw1
    L1 = Hp1 * Wp1
    Lacc1 = (H - 1) * Wp1 + W
    kh2, kw2, ph2, pw2 = 3, 3, 1, 1
    Hp2, Wp2 = H + 2 * ph2, W + 2 * pw2
    L2 = Hp2 * Wp2
    Lacc2 = (H - 1) * Wp2 + W

    def body(x_ref, xs_ref, w12a_ref, b12a_ref, wp1_ref, b1b_ref,
             w2s_ref, b2b_ref, w3_ref, b3_ref, o_ref,
             z1_ref, z2_ref):
        f32, bf16 = jnp.float32, jnp.bfloat16
        x = x_ref[0]
        xs = xs_ref[0]

        m = jnp.max(x, axis=1, keepdims=True)
        x2 = jnp.concatenate([jnp.broadcast_to(m, (C, HW)), x], axis=0)
        x2s = jnp.concatenate([jnp.broadcast_to(m, (C, nS)), xs], axis=0)
        xin = jnp.concatenate([x2, x2s], axis=1).astype(bf16)
        b12a = b12a_ref[...]
        y12 = jnp.dot(w12a_ref[...], xin, preferred_element_type=f32) + b12a
        y12 = jnp.maximum(y12, 0.0)
        y2a = y12[n1:, :HW]
        y1as = y12[:n1, HW:]

        z1_ref[...] = jnp.zeros_like(z1_ref)
        rb1a = jnp.maximum(b12a[:n1], 0.0).astype(bf16)
        for r in range(Ho1a):
            st = (ph1 + r) * Wp1 + pw1
            z1_ref[:n1, st:st + Wo1a] = jnp.broadcast_to(rb1a, (n1, Wo1a))
        for r in range(nr):
            st = (ph1 + i_lo + r) * Wp1 + (pw1 + j_lo)
            z1_ref[:n1, st:st + nc] = y1as[:, r * nc:(r + 1) * nc].astype(bf16)
        z1_ref[n1:, :L1 - kw1 // 2] = z1_ref[:n1, kw1 // 2:]

        acc1 = jnp.broadcast_to(b1b_ref[...], (n1, Lacc1)).astype(f32)
        for di in range(kh1):
            base = di * Wp1
            for dj in range(kw1 // 2):
                acc1 = acc1 + jnp.dot(wp1_ref[di * 5 + dj],
                                      z1_ref[:, base + dj:base + dj + Lacc1],
                                      preferred_element_type=f32)
        h1 = jnp.maximum(acc1, 0.0)

        z2_ref[...] = jnp.zeros_like(z2_ref)
        for r in range(H):
            st = (ph2 + r) * Wp2 + pw2
            z2_ref[:n2, st:st + W] = y2a[:, r * W:(r + 1) * W].astype(bf16)
        z2_ref[n2:2 * n2, :L2 - 1] = z2_ref[:n2, 1:]
        z2_ref[2 * n2:, :L2 - 2] = z2_ref[:n2, 2:]

        acc2 = jnp.broadcast_to(b2b_ref[...], (n2, Lacc2)).astype(f32)
        for di in range(kh2):
            acc2 = acc2 + jnp.dot(
                w2s_ref[di], z2_ref[:, di * Wp2:di * Wp2 + Lacc2],
                preferred_element_type=f32)
        h2 = jnp.maximum(acc2, 0.0)

        h1v = jnp.concatenate([h1[:, r * Wp1:r * Wp1 + W] for r in range(H)],
                              axis=1)
        h2v = jnp.concatenate([h2[:, r * Wp2:r * Wp2 + W] for r in range(H)],
                              axis=1)
        cat = jnp.concatenate([h1v, h2v], axis=0).astype(bf16)
        h3 = jnp.dot(w3_ref[...], cat, preferred_element_type=f32) + b3_ref[...]
        o_ref[0, :n3] = jnp.maximum(h3, 0.0)
        o_ref[0, n3:] = x

    geom = dict(HW=HW, nS=nS, L1=L1, L2=L2, Lacc1=Lacc1, Lacc2=Lacc2,
                kh1=kh1, kw1=kw1, kh2=kh2, kw2=kw2)
    return body, geom


def kernel(x, w1a, b1a, w1b, b1b, w2a, b2a, w2b, b2b, w3, b3):
    B, C, H, W = x.shape
    n1, n2, n3 = w1b.shape[0], w2b.shape[0], w3.shape[0]
    body, g = _make_body(C, H, W, n1, n2, n3)
    HW, nS, L1, L2 = g["HW"], g["nS"], g["L1"], g["L2"]
    bf16, f32 = jnp.bfloat16, jnp.float32

    x_cm = x.reshape(B, C, HW)
    xs = x[:, :, ::2, ::2].reshape(B, C, nS)
    w12a = jnp.concatenate([w1a.reshape(n1, 2 * C),
                            w2a.reshape(n2, 2 * C)], axis=0).astype(bf16)
    b12a = jnp.concatenate([b1a, b2a]).reshape(n1 + n2, 1).astype(f32)
    w1t = w1b.transpose(2, 3, 0, 1)
    wp1 = jnp.stack([jnp.concatenate([w1t[di, dj], w1t[di, dj + 5]], axis=1)
                     for di in range(g["kh1"]) for dj in range(g["kw1"] // 2)]
                    ).astype(bf16)
    w2t = w2b.transpose(2, 3, 0, 1)
    w2s = jnp.stack([jnp.concatenate([w2t[di, 0], w2t[di, 1], w2t[di, 2]],
                                     axis=1) for di in range(g["kh2"])]
                    ).astype(bf16)
    w3m = w3.reshape(n3, n1 + n2).astype(bf16)
    b1bm = b1b.reshape(n1, 1).astype(f32)
    b2bm = b2b.reshape(n2, 1).astype(f32)
    b3m = b3.reshape(n3, 1).astype(f32)

    out = pl.pallas_call(
        body,
        out_shape=jax.ShapeDtypeStruct((B, n3 + C, HW), f32),
        grid=(B,),
        in_specs=[
            pl.BlockSpec((1, C, HW), lambda b: (b, 0, 0)),
            pl.BlockSpec((1, C, nS), lambda b: (b, 0, 0)),
            pl.BlockSpec((n1 + n2, 2 * C), lambda b: (0, 0)),
            pl.BlockSpec((n1 + n2, 1), lambda b: (0, 0)),
            pl.BlockSpec((50, n1, 2 * n1), lambda b: (0, 0, 0)),
            pl.BlockSpec((n1, 1), lambda b: (0, 0)),
            pl.BlockSpec((3, n2, 3 * n2), lambda b: (0, 0, 0)),
            pl.BlockSpec((n2, 1), lambda b: (0, 0)),
            pl.BlockSpec((n3, n1 + n2), lambda b: (0, 0)),
            pl.BlockSpec((n3, 1), lambda b: (0, 0)),
        ],
        out_specs=pl.BlockSpec((1, n3 + C, HW), lambda b: (b, 0, 0)),
        scratch_shapes=[pltpu.VMEM((2 * n1, L1), bf16),
                        pltpu.VMEM((3 * n2, L2), bf16)],
        compiler_params=pltpu.CompilerParams(dimension_semantics=("parallel",)),
    )(x_cm, xs, w12a, b12a, wp1, b1bm, w2s, b2bm, w3m, b3m)
    return out.reshape(B, n3 + C, H, W)
